# Initial kernel scaffold; baseline (speedup 1.0000x reference)
import jax
import jax.numpy as jnp
from jax.experimental import pallas as pl


def kernel(x, edge_index, params):
    raise NotImplementedError("write your pallas kernel here")


# speedup vs baseline: 6.9409x; 6.9409x over previous
import functools

import jax
import jax.numpy as jnp
from jax import lax
from jax.experimental import pallas as pl
from jax.experimental.pallas import tpu as pltpu
from jax.experimental.pallas import tpu_sc as plsc

N = 10000
E = 320000
D = 128

NC = 2
NS = 16
NW = NC * NS
EPT = E // NW
CHUNK = 80
NCHUNK = EPT // CHUNK
RPT = 624
REM = N - NS * RPT


def _sc_agg_body(h_hbm, row_hbm, col_hbm, zeros_hbm, out_hbm,
                 row_v, col_v, msg_v, acc_sh, sem):
    c = lax.axis_index("c")
    s = lax.axis_index("s")
    wid = c * NS + s

    pltpu.sync_copy(zeros_hbm.at[pl.ds(s * RPT, RPT)],
                    acc_sh.at[pl.ds(s * RPT, RPT)])

    @pl.when(s == 0)
    def _zero_rem():
        pltpu.sync_copy(zeros_hbm.at[pl.ds(NS * RPT, REM)],
                        acc_sh.at[pl.ds(NS * RPT, REM)])
    pltpu.sync_copy(row_hbm.at[wid], row_v)
    pltpu.sync_copy(col_hbm.at[wid], col_v)
    plsc.subcore_barrier()

    @pl.loop(0, NCHUNK)
    def _chunk(j):
        pltpu.async_copy(h_hbm.at[row_v.at[j]], msg_v, sem).wait()
        pltpu.sync_copy(msg_v, acc_sh.at[col_v.at[j]], add=True)

    plsc.subcore_barrier()
    pltpu.sync_copy(acc_sh.at[pl.ds(s * RPT, RPT)],
                    out_hbm.at[c, pl.ds(s * RPT, RPT)])

    @pl.when(s == 0)
    def _drain_rem():
        pltpu.sync_copy(acc_sh.at[pl.ds(NS * RPT, REM)],
                        out_hbm.at[c, pl.ds(NS * RPT, REM)])


@jax.jit
def _sc_agg(h, row2d, col2d, zeros):
    mesh = plsc.VectorSubcoreMesh(core_axis_name="c", subcore_axis_name="s",
                                  num_cores=NC, num_subcores=NS)
    f = pl.kernel(
        _sc_agg_body,
        out_type=jax.ShapeDtypeStruct((NC, N, D), jnp.float32),
        mesh=mesh,
        scratch_types=[
            pltpu.VMEM((NCHUNK, CHUNK), jnp.int32),
            pltpu.VMEM((NCHUNK, CHUNK), jnp.int32),
            pltpu.VMEM((CHUNK, D), jnp.float32),
            pltpu.VMEM_SHARED((N, D), jnp.float32),
            pltpu.SemaphoreType.DMA,
        ],
    )
    return f(h, row2d, col2d, zeros)


def _fc_body(x_ref, w_ref, h_ref, hr_ref):
    h = lax.dot_general(x_ref[...], w_ref[...], (((1,), (1,)), ((), ())),
                        preferred_element_type=jnp.float32)
    h_ref[...] = h
    hr_ref[...] = jnp.maximum(h, 0.0)


@jax.jit
def _fc(x, w_fc):
    return pl.pallas_call(
        _fc_body,
        out_shape=(jax.ShapeDtypeStruct((N, D), jnp.float32),
                   jax.ShapeDtypeStruct((N, D), jnp.float32)),
    )(x, w_fc)


def _bn(y, g, b):
    m = jnp.mean(y, axis=0, keepdims=True)
    v = jnp.mean((y - m) * (y - m), axis=0, keepdims=True)
    return g * (y - m) / jnp.sqrt(v + 1e-5) + b


def _mlp_body(final_relu, h_ref, agg_ref, eps_ref, w1_ref, b1_ref, g1_ref,
              be1_ref, w2_ref, b2_ref, gbn_ref, bbn_ref, out_ref):
    h = h_ref[...]
    z = (1.0 + eps_ref[0, 0]) * h + agg_ref[0] + agg_ref[1]
    y = lax.dot_general(z, w1_ref[...], (((1,), (1,)), ((), ())),
                        preferred_element_type=jnp.float32) + b1_ref[...]
    y = jnp.maximum(_bn(y, g1_ref[...], be1_ref[...]), 0.0)
    o = lax.dot_general(y, w2_ref[...], (((1,), (1,)), ((), ())),
                        preferred_element_type=jnp.float32) + b2_ref[...]
    o = _bn(o, gbn_ref[...], bbn_ref[...])
    if final_relu:
        o = jnp.maximum(o, 0.0)
    out_ref[...] = o


@functools.partial(jax.jit, static_argnames=("final_relu",))
def _mlp(h, agg, lp, final_relu):
    body = functools.partial(_mlp_body, final_relu)
    return pl.pallas_call(
        body,
        out_shape=jax.ShapeDtypeStruct((N, D), jnp.float32),
    )(h, agg, lp['eps'].reshape(1, 1),
      lp['W1'], lp['b1'].reshape(1, 2 * D), lp['g1'].reshape(1, 2 * D),
      lp['be1'].reshape(1, 2 * D),
      lp['W2'], lp['b2'].reshape(1, D), lp['gbn'].reshape(1, D),
      lp['bbn'].reshape(1, D))


def kernel(x, edge_index, params):
    row = edge_index[0].astype(jnp.int32).reshape(NW, NCHUNK, CHUNK)
    col = edge_index[1].astype(jnp.int32).reshape(NW, NCHUNK, CHUNK)
    zeros = jnp.zeros((N, D), jnp.float32)

    h, hsrc = _fc(x, params['W_fc'])
    outs = []
    for li, lp in enumerate(params['layers']):
        agg = _sc_agg(hsrc, row, col, zeros)
        h = _mlp(h, agg, lp, final_relu=li < 2)
        hsrc = h
        outs.append(h)
    return (outs[2], outs[0])


# speedup vs baseline: 9.9814x; 1.4380x over previous
import functools

import jax
import jax.numpy as jnp
from jax import lax
from jax.experimental import pallas as pl
from jax.experimental.pallas import tpu as pltpu
from jax.experimental.pallas import tpu_sc as plsc

N = 10000
E = 320000
D = 128

NC = 2
NS = 16
NW = NC * NS
EPT = E // NW
CHUNK = 80
NCHUNK = EPT // CHUNK
RPT = 624
REM = N - NS * RPT
NBUF = 3
KAHEAD = NBUF - 1


def _sc_agg_body(h_hbm, row_hbm, col_hbm, zeros_hbm, out_hbm,
                 irow_v, icol_v, msg_v, acc_sh, sems):
    c = lax.axis_index("c")
    s = lax.axis_index("s")
    wid = c * NS + s
    ebase = wid * EPT

    pltpu.sync_copy(zeros_hbm.at[pl.ds(s * RPT, RPT)],
                    acc_sh.at[pl.ds(s * RPT, RPT)])

    @pl.when(s == 0)
    def _zero_rem():
        pltpu.sync_copy(zeros_hbm.at[pl.ds(NS * RPT, REM)],
                        acc_sh.at[pl.ds(NS * RPT, REM)])

    plsc.subcore_barrier()

    gsem, ssem, isem = sems

    def load_idx(j, b):
        off = ebase + j * CHUNK
        pltpu.async_copy(row_hbm.at[pl.ds(off, CHUNK)], irow_v.at[b], isem)
        pltpu.async_copy(col_hbm.at[pl.ds(off, CHUNK)], icol_v.at[b], isem)

    def wait_idx(b):
        pltpu.make_async_copy(row_hbm.at[pl.ds(0, CHUNK)], irow_v.at[b],
                              isem).wait()
        pltpu.make_async_copy(col_hbm.at[pl.ds(0, CHUNK)], icol_v.at[b],
                              isem).wait()

    def start_gather(b):
        pltpu.async_copy(h_hbm.at[irow_v.at[b]], msg_v.at[b], gsem)

    for b in range(NBUF):
        load_idx(b, b)
    for k in range(KAHEAD):
        wait_idx(k)
        start_gather(k)

    @pl.loop(0, NCHUNK)
    def _chunk(j):
        b = lax.rem(j, NBUF)

        @pl.when(j + KAHEAD < NCHUNK)
        def _gather_ahead():
            bk = lax.rem(j + KAHEAD, NBUF)
            wait_idx(bk)
            start_gather(bk)

        pltpu.make_async_copy(h_hbm.at[pl.ds(0, CHUNK)], msg_v.at[b],
                              gsem).wait()
        pltpu.async_copy(msg_v.at[b], acc_sh.at[icol_v.at[b]], ssem,
                         add=True).wait()

        @pl.when(j + NBUF < NCHUNK)
        def _refill():
            load_idx(j + NBUF, b)

    plsc.subcore_barrier()
    pltpu.sync_copy(acc_sh.at[pl.ds(s * RPT, RPT)],
                    out_hbm.at[c, pl.ds(s * RPT, RPT)])

    @pl.when(s == 0)
    def _drain_rem():
        pltpu.sync_copy(acc_sh.at[pl.ds(NS * RPT, REM)],
                        out_hbm.at[c, pl.ds(NS * RPT, REM)])


@jax.jit
def _sc_agg(h, row, col, zeros):
    mesh = plsc.VectorSubcoreMesh(core_axis_name="c", subcore_axis_name="s",
                                  num_cores=NC, num_subcores=NS)
    f = pl.kernel(
        _sc_agg_body,
        out_type=jax.ShapeDtypeStruct((NC, N, D), jnp.float32),
        mesh=mesh,
        scratch_types=[
            pltpu.VMEM((NBUF, CHUNK), jnp.int32),
            pltpu.VMEM((NBUF, CHUNK), jnp.int32),
            pltpu.VMEM((NBUF, CHUNK, D), jnp.float32),
            pltpu.VMEM_SHARED((N, D), jnp.float32),
            (pltpu.SemaphoreType.DMA, pltpu.SemaphoreType.DMA,
             pltpu.SemaphoreType.DMA),
        ],
    )
    return f(h, row, col, zeros)


def _fc_body(x_ref, w_ref, h_ref, hr_ref):
    h = lax.dot_general(x_ref[...], w_ref[...], (((1,), (1,)), ((), ())),
                        preferred_element_type=jnp.float32)
    h_ref[...] = h
    hr_ref[...] = jnp.maximum(h, 0.0)


@jax.jit
def _fc(x, w_fc):
    return pl.pallas_call(
        _fc_body,
        out_shape=(jax.ShapeDtypeStruct((N, D), jnp.float32),
                   jax.ShapeDtypeStruct((N, D), jnp.float32)),
    )(x, w_fc)


def _bn(y, g, b):
    m = jnp.mean(y, axis=0, keepdims=True)
    v = jnp.mean((y - m) * (y - m), axis=0, keepdims=True)
    return g * (y - m) / jnp.sqrt(v + 1e-5) + b


def _mlp_body(final_relu, h_ref, agg_ref, eps_ref, w1_ref, b1_ref, g1_ref,
              be1_ref, w2_ref, b2_ref, gbn_ref, bbn_ref, out_ref):
    h = h_ref[...]
    z = (1.0 + eps_ref[0, 0]) * h + agg_ref[0] + agg_ref[1]
    y = lax.dot_general(z, w1_ref[...], (((1,), (1,)), ((), ())),
                        preferred_element_type=jnp.float32) + b1_ref[...]
    y = jnp.maximum(_bn(y, g1_ref[...], be1_ref[...]), 0.0)
    o = lax.dot_general(y, w2_ref[...], (((1,), (1,)), ((), ())),
                        preferred_element_type=jnp.float32) + b2_ref[...]
    o = _bn(o, gbn_ref[...], bbn_ref[...])
    if final_relu:
        o = jnp.maximum(o, 0.0)
    out_ref[...] = o


@functools.partial(jax.jit, static_argnames=("final_relu",))
def _mlp(h, agg, lp, final_relu):
    body = functools.partial(_mlp_body, final_relu)
    return pl.pallas_call(
        body,
        out_shape=jax.ShapeDtypeStruct((N, D), jnp.float32),
    )(h, agg, lp['eps'].reshape(1, 1),
      lp['W1'], lp['b1'].reshape(1, 2 * D), lp['g1'].reshape(1, 2 * D),
      lp['be1'].reshape(1, 2 * D),
      lp['W2'], lp['b2'].reshape(1, D), lp['gbn'].reshape(1, D),
      lp['bbn'].reshape(1, D))


def kernel(x, edge_index, params):
    row = edge_index[0].astype(jnp.int32)
    col = edge_index[1].astype(jnp.int32)
    zeros = jnp.zeros((N, D), jnp.float32)

    h, hsrc = _fc(x, params['W_fc'])
    outs = []
    for li, lp in enumerate(params['layers']):
        agg = _sc_agg(hsrc, row, col, zeros)
        h = _mlp(h, agg, lp, final_relu=li < 2)
        hsrc = h
        outs.append(h)
    return (outs[2], outs[0])


# speedup vs baseline: 13.1323x; 1.3157x over previous
import functools

import jax
import jax.numpy as jnp
from jax import lax
from jax.experimental import pallas as pl
from jax.experimental.pallas import tpu as pltpu
from jax.experimental.pallas import tpu_sc as plsc

N = 10000
E = 320000
D = 128

NC = 2
NS = 16
NW = NC * NS
EPT = E // NW
CHUNK = 80
NCHUNK = EPT // CHUNK
RPT = 624
REM = N - NS * RPT
NBUF = 3
KAHEAD = NBUF - 1
IBUF = NBUF + 2


def _sc_agg_body(h_hbm, row_hbm, col_hbm, zeros_hbm, out_hbm,
                 irow_v, icol_v, msg_v, acc_sh, sems):
    c = lax.axis_index("c")
    s = lax.axis_index("s")
    wid = c * NS + s
    ebase = wid * EPT

    pltpu.sync_copy(zeros_hbm.at[pl.ds(s * RPT, RPT)],
                    acc_sh.at[pl.ds(s * RPT, RPT)])

    @pl.when(s == 0)
    def _zero_rem():
        pltpu.sync_copy(zeros_hbm.at[pl.ds(NS * RPT, REM)],
                        acc_sh.at[pl.ds(NS * RPT, REM)])

    plsc.subcore_barrier()

    gsem, ssem, isem = sems

    def load_idx(j, b):
        off = ebase + j * CHUNK
        pltpu.async_copy(row_hbm.at[pl.ds(off, CHUNK)], irow_v.at[b], isem)
        pltpu.async_copy(col_hbm.at[pl.ds(off, CHUNK)], icol_v.at[b], isem)

    def wait_idx(b):
        pltpu.make_async_copy(row_hbm.at[pl.ds(0, CHUNK)], irow_v.at[b],
                              isem).wait()
        pltpu.make_async_copy(col_hbm.at[pl.ds(0, CHUNK)], icol_v.at[b],
                              isem).wait()

    def start_gather(bm, bi):
        pltpu.async_copy(h_hbm.at[irow_v.at[bi]], msg_v.at[bm], gsem)

    def wait_scatter():
        pltpu.make_async_copy(h_hbm.at[pl.ds(0, CHUNK)], msg_v.at[0],
                              ssem).wait()

    for b in range(IBUF):
        load_idx(b, b)
    for k in range(KAHEAD):
        wait_idx(k)
        start_gather(k, k)

    @pl.loop(0, NCHUNK)
    def _chunk(j):
        b = lax.rem(j, NBUF)

        @pl.when(j + KAHEAD < NCHUNK)
        def _gather_ahead():
            bk = lax.rem(j + KAHEAD, NBUF)
            @pl.when(j >= 1)
            def _drain_prev():
                wait_scatter()

                @pl.when(j - 1 + IBUF < NCHUNK)
                def _refill_idx():
                    load_idx(j - 1 + IBUF, lax.rem(j - 1, IBUF))

            wait_idx(lax.rem(j + KAHEAD, IBUF))
            start_gather(bk, lax.rem(j + KAHEAD, IBUF))

        pltpu.make_async_copy(h_hbm.at[pl.ds(0, CHUNK)], msg_v.at[b],
                              gsem).wait()
        pltpu.async_copy(msg_v.at[b], acc_sh.at[icol_v.at[lax.rem(j, IBUF)]],
                         ssem, add=True)

    for _ in range(KAHEAD + 1):
        wait_scatter()

    plsc.subcore_barrier()
    pltpu.sync_copy(acc_sh.at[pl.ds(s * RPT, RPT)],
                    out_hbm.at[c, pl.ds(s * RPT, RPT)])

    @pl.when(s == 0)
    def _drain_rem():
        pltpu.sync_copy(acc_sh.at[pl.ds(NS * RPT, REM)],
                        out_hbm.at[c, pl.ds(NS * RPT, REM)])


@jax.jit
def _sc_agg(h, row, col, zeros):
    mesh = plsc.VectorSubcoreMesh(core_axis_name="c", subcore_axis_name="s",
                                  num_cores=NC, num_subcores=NS)
    f = pl.kernel(
        _sc_agg_body,
        out_type=jax.ShapeDtypeStruct((NC, N, D), jnp.float32),
        mesh=mesh,
        scratch_types=[
            pltpu.VMEM((IBUF, CHUNK), jnp.int32),
            pltpu.VMEM((IBUF, CHUNK), jnp.int32),
            pltpu.VMEM((NBUF, CHUNK, D), jnp.float32),
            pltpu.VMEM_SHARED((N, D), jnp.float32),
            (pltpu.SemaphoreType.DMA, pltpu.SemaphoreType.DMA,
             pltpu.SemaphoreType.DMA),
        ],
    )
    return f(h, row, col, zeros)


def _fc_body(x_ref, w_ref, h_ref, hr_ref):
    h = lax.dot_general(x_ref[...], w_ref[...], (((1,), (1,)), ((), ())),
                        preferred_element_type=jnp.float32)
    h_ref[...] = h
    hr_ref[...] = jnp.maximum(h, 0.0)


@jax.jit
def _fc(x, w_fc):
    return pl.pallas_call(
        _fc_body,
        out_shape=(jax.ShapeDtypeStruct((N, D), jnp.float32),
                   jax.ShapeDtypeStruct((N, D), jnp.float32)),
    )(x, w_fc)


def _bn(y, g, b):
    m = jnp.mean(y, axis=0, keepdims=True)
    v = jnp.mean((y - m) * (y - m), axis=0, keepdims=True)
    return g * (y - m) / jnp.sqrt(v + 1e-5) + b


def _mlp_body(final_relu, h_ref, agg_ref, eps_ref, w1_ref, b1_ref, g1_ref,
              be1_ref, w2_ref, b2_ref, gbn_ref, bbn_ref, out_ref):
    h = h_ref[...]
    z = (1.0 + eps_ref[0, 0]) * h + agg_ref[0] + agg_ref[1]
    y = lax.dot_general(z, w1_ref[...], (((1,), (1,)), ((), ())),
                        preferred_element_type=jnp.float32) + b1_ref[...]
    y = jnp.maximum(_bn(y, g1_ref[...], be1_ref[...]), 0.0)
    o = lax.dot_general(y, w2_ref[...], (((1,), (1,)), ((), ())),
                        preferred_element_type=jnp.float32) + b2_ref[...]
    o = _bn(o, gbn_ref[...], bbn_ref[...])
    if final_relu:
        o = jnp.maximum(o, 0.0)
    out_ref[...] = o


@functools.partial(jax.jit, static_argnames=("final_relu",))
def _mlp(h, agg, lp, final_relu):
    body = functools.partial(_mlp_body, final_relu)
    return pl.pallas_call(
        body,
        out_shape=jax.ShapeDtypeStruct((N, D), jnp.float32),
    )(h, agg, lp['eps'].reshape(1, 1),
      lp['W1'], lp['b1'].reshape(1, 2 * D), lp['g1'].reshape(1, 2 * D),
      lp['be1'].reshape(1, 2 * D),
      lp['W2'], lp['b2'].reshape(1, D), lp['gbn'].reshape(1, D),
      lp['bbn'].reshape(1, D))


def kernel(x, edge_index, params):
    row = edge_index[0].astype(jnp.int32)
    col = edge_index[1].astype(jnp.int32)
    zeros = jnp.zeros((N, D), jnp.float32)

    h, hsrc = _fc(x, params['W_fc'])
    outs = []
    for li, lp in enumerate(params['layers']):
        agg = _sc_agg(hsrc, row, col, zeros)
        h = _mlp(h, agg, lp, final_relu=li < 2)
        hsrc = h
        outs.append(h)
    return (outs[2], outs[0])


# speedup vs baseline: 13.7917x; 1.0502x over previous
import functools

import jax
import jax.numpy as jnp
from jax import lax
from jax.experimental import pallas as pl
from jax.experimental.pallas import tpu as pltpu
from jax.experimental.pallas import tpu_sc as plsc

N = 10000
E = 320000
D = 128

NC = 2
NS = 16
NW = NC * NS
EPT = E // NW
CHUNK = 80
NCHUNK = EPT // CHUNK
RPT = 624
REM = N - NS * RPT
NBUF = 3
KAHEAD = NBUF - 1
IBUF = NBUF + 2


def _sc_agg_body(h_hbm, row_hbm, col_hbm, zeros_hbm, out_hbm,
                 irow_v, icol_v, msg_v, acc_sh, sems):
    c = lax.axis_index("c")
    s = lax.axis_index("s")
    wid = c * NS + s
    ebase = wid * EPT

    gsem, ssem, isem = sems

    def load_idx(j, b):
        off = ebase + j * CHUNK
        pltpu.async_copy(row_hbm.at[pl.ds(off, CHUNK)], irow_v.at[b], isem)
        pltpu.async_copy(col_hbm.at[pl.ds(off, CHUNK)], icol_v.at[b], isem)

    def wait_idx(b):
        pltpu.make_async_copy(row_hbm.at[pl.ds(0, CHUNK)], irow_v.at[b],
                              isem).wait()
        pltpu.make_async_copy(col_hbm.at[pl.ds(0, CHUNK)], icol_v.at[b],
                              isem).wait()

    def start_gather(bm, bi):
        pltpu.async_copy(h_hbm.at[irow_v.at[bi]], msg_v.at[bm], gsem)

    def wait_scatter():
        pltpu.make_async_copy(h_hbm.at[pl.ds(0, CHUNK)], msg_v.at[0],
                              ssem).wait()

    for b in range(IBUF):
        load_idx(b, b)
    for k in range(KAHEAD):
        wait_idx(k)
        start_gather(k, k)

    pltpu.sync_copy(zeros_hbm.at[pl.ds(s * RPT, RPT)],
                    acc_sh.at[pl.ds(s * RPT, RPT)])

    @pl.when(s == 0)
    def _zero_rem():
        pltpu.sync_copy(zeros_hbm.at[pl.ds(NS * RPT, REM)],
                        acc_sh.at[pl.ds(NS * RPT, REM)])

    plsc.subcore_barrier()

    @pl.loop(0, NCHUNK)
    def _chunk(j):
        b = lax.rem(j, NBUF)

        @pl.when(j + KAHEAD < NCHUNK)
        def _gather_ahead():
            bk = lax.rem(j + KAHEAD, NBUF)
            @pl.when(j >= 1)
            def _drain_prev():
                wait_scatter()

                @pl.when(j - 1 + IBUF < NCHUNK)
                def _refill_idx():
                    load_idx(j - 1 + IBUF, lax.rem(j - 1, IBUF))

            wait_idx(lax.rem(j + KAHEAD, IBUF))
            start_gather(bk, lax.rem(j + KAHEAD, IBUF))

        pltpu.make_async_copy(h_hbm.at[pl.ds(0, CHUNK)], msg_v.at[b],
                              gsem).wait()
        pltpu.async_copy(msg_v.at[b], acc_sh.at[icol_v.at[lax.rem(j, IBUF)]],
                         ssem, add=True)

    for _ in range(KAHEAD + 1):
        wait_scatter()

    plsc.subcore_barrier()
    pltpu.sync_copy(acc_sh.at[pl.ds(s * RPT, RPT)],
                    out_hbm.at[c, pl.ds(s * RPT, RPT)])

    @pl.when(s == 0)
    def _drain_rem():
        pltpu.sync_copy(acc_sh.at[pl.ds(NS * RPT, REM)],
                        out_hbm.at[c, pl.ds(NS * RPT, REM)])


@jax.jit
def _sc_agg(h, row, col, zeros):
    mesh = plsc.VectorSubcoreMesh(core_axis_name="c", subcore_axis_name="s",
                                  num_cores=NC, num_subcores=NS)
    f = pl.kernel(
        _sc_agg_body,
        out_type=jax.ShapeDtypeStruct((NC, N, D), jnp.float32),
        mesh=mesh,
        scratch_types=[
            pltpu.VMEM((IBUF, CHUNK), jnp.int32),
            pltpu.VMEM((IBUF, CHUNK), jnp.int32),
            pltpu.VMEM((NBUF, CHUNK, D), jnp.float32),
            pltpu.VMEM_SHARED((N, D), jnp.float32),
            (pltpu.SemaphoreType.DMA, pltpu.SemaphoreType.DMA,
             pltpu.SemaphoreType.DMA),
        ],
    )
    return f(h, row, col, zeros)


def _fc_body(x_ref, w_ref, h_ref, hr_ref):
    h = lax.dot_general(x_ref[...], w_ref[...], (((1,), (1,)), ((), ())),
                        preferred_element_type=jnp.float32)
    h_ref[...] = h
    hr_ref[...] = jnp.maximum(h, 0.0)


@jax.jit
def _fc(x, w_fc):
    return pl.pallas_call(
        _fc_body,
        out_shape=(jax.ShapeDtypeStruct((N, D), jnp.float32),
                   jax.ShapeDtypeStruct((N, D), jnp.float32)),
    )(x, w_fc)


def _bn(y, g, b):
    n_inv = 1.0 / y.shape[0]
    m = jnp.sum(y, axis=0, keepdims=True) * n_inv
    v = jnp.sum(y * y, axis=0, keepdims=True) * n_inv - m * m
    scale = g * lax.rsqrt(v + 1e-5)
    return y * scale + (b - m * scale)


def _mlp_body(final_relu, h_ref, agg_ref, eps_ref, w1_ref, g1_ref,
              be1_ref, w2_ref, gbn_ref, bbn_ref, out_ref):
    h = h_ref[...]
    z = (1.0 + eps_ref[0, 0]) * h + agg_ref[0] + agg_ref[1]
    y = lax.dot_general(z, w1_ref[...], (((1,), (1,)), ((), ())),
                        preferred_element_type=jnp.float32)
    y = jnp.maximum(_bn(y, g1_ref[...], be1_ref[...]), 0.0)
    o = lax.dot_general(y, w2_ref[...], (((1,), (1,)), ((), ())),
                        preferred_element_type=jnp.float32)
    o = _bn(o, gbn_ref[...], bbn_ref[...])
    if final_relu:
        o = jnp.maximum(o, 0.0)
    out_ref[...] = o


@functools.partial(jax.jit, static_argnames=("final_relu",))
def _mlp(h, agg, lp, final_relu):
    body = functools.partial(_mlp_body, final_relu)
    return pl.pallas_call(
        body,
        out_shape=jax.ShapeDtypeStruct((N, D), jnp.float32),
    )(h, agg, lp['eps'].reshape(1, 1),
      lp['W1'], lp['g1'].reshape(1, 2 * D), lp['be1'].reshape(1, 2 * D),
      lp['W2'], lp['gbn'].reshape(1, D), lp['bbn'].reshape(1, D))


def kernel(x, edge_index, params):
    row = edge_index[0].astype(jnp.int32)
    col = edge_index[1].astype(jnp.int32)
    zeros = jnp.zeros((N, D), jnp.float32)

    h, hsrc = _fc(x, params['W_fc'])
    outs = []
    for li, lp in enumerate(params['layers']):
        agg = _sc_agg(hsrc, row, col, zeros)
        h = _mlp(h, agg, lp, final_relu=li < 2)
        hsrc = h
        outs.append(h)
    return (outs[2], outs[0])
